# baseline (speedup 1.0000x reference)
import jax
import jax.numpy as jnp
from jax.experimental import pallas as pl
from jax.experimental.pallas import tpu as pltpu


def kernel(frames):
    B, T, H, W = frames.shape
    Ts = T // 4
    idx = [(21 * p) // 5 for p in range(Ts)]

    def body(in_ref, slow_ref, fast_ref, sem_f, sem_s):
        fast_cp = pltpu.make_async_copy(in_ref, fast_ref, sem_f)
        fast_cp.start()
        slow_cps = []
        for b in range(B):
            for p, t in enumerate(idx):
                c = pltpu.make_async_copy(
                    in_ref.at[b, t], slow_ref.at[b, p], sem_s
                )
                c.start()
                slow_cps.append(c)
        for c in slow_cps:
            c.wait()
        fast_cp.wait()

    slow, fast = pl.pallas_call(
        body,
        in_specs=[pl.BlockSpec(memory_space=pl.ANY)],
        out_specs=(
            pl.BlockSpec(memory_space=pl.ANY),
            pl.BlockSpec(memory_space=pl.ANY),
        ),
        out_shape=(
            jax.ShapeDtypeStruct((B, Ts, H, W), frames.dtype),
            jax.ShapeDtypeStruct((B, T, H, W), frames.dtype),
        ),
        scratch_shapes=[pltpu.SemaphoreType.DMA, pltpu.SemaphoreType.DMA],
    )(frames)
    return (slow, fast)


# speedup vs baseline: 36.2621x; 36.2621x over previous
import jax
import jax.numpy as jnp
from jax.experimental import pallas as pl
from jax.experimental.pallas import tpu as pltpu


def _body(in_ref, slow_ref, fast_ref):
    p = pl.program_id(1)
    fast_ref[...] = in_ref[...]
    o = p // 5
    slow_ref[...] = in_ref[:, pl.ds(o, 1)]


def kernel(frames):
    B, T, H, W = frames.shape
    Ts = T // 4

    slow, fast = pl.pallas_call(
        _body,
        grid=(B, Ts),
        in_specs=[pl.BlockSpec((1, 4, H, W), lambda b, p: (b, p, 0, 0))],
        out_specs=(
            pl.BlockSpec((1, 1, H, W), lambda b, p: (b, p, 0, 0)),
            pl.BlockSpec((1, 4, H, W), lambda b, p: (b, p, 0, 0)),
        ),
        out_shape=(
            jax.ShapeDtypeStruct((B, Ts, H, W), frames.dtype),
            jax.ShapeDtypeStruct((B, T, H, W), frames.dtype),
        ),
        compiler_params=pltpu.CompilerParams(
            dimension_semantics=("parallel", "parallel"),
        ),
    )(frames)
    return (slow, fast)


# speedup vs baseline: 47.6283x; 1.3134x over previous
import jax
import jax.numpy as jnp
from jax.experimental import pallas as pl
from jax.experimental.pallas import tpu as pltpu

_CF = 8
_SPC = _CF // 4


def _body(in_ref, slow_ref, fast_ref):
    q = pl.program_id(1)
    fast_ref[...] = in_ref[...]
    for j in range(_SPC):
        p = _SPC * q + j
        o = (21 * p) // 5 - _CF * q
        slow_ref[:, j : j + 1] = in_ref[:, pl.ds(o, 1)]


def kernel(frames):
    B, T, H, W = frames.shape
    Ts = T // 4

    slow, fast = pl.pallas_call(
        _body,
        grid=(B, T // _CF),
        in_specs=[pl.BlockSpec((1, _CF, H, W), lambda b, q: (b, q, 0, 0))],
        out_specs=(
            pl.BlockSpec((1, _SPC, H, W), lambda b, q: (b, q, 0, 0)),
            pl.BlockSpec((1, _CF, H, W), lambda b, q: (b, q, 0, 0)),
        ),
        out_shape=(
            jax.ShapeDtypeStruct((B, Ts, H, W), frames.dtype),
            jax.ShapeDtypeStruct((B, T, H, W), frames.dtype),
        ),
        compiler_params=pltpu.CompilerParams(
            dimension_semantics=("parallel", "parallel"),
        ),
    )(frames)
    return (slow, fast)


# speedup vs baseline: 51.1157x; 1.0732x over previous
import jax
import jax.numpy as jnp
from jax.experimental import pallas as pl
from jax.experimental.pallas import tpu as pltpu

_CF = 16
_SPC = _CF // 4


def _body(in_ref, slow_ref, fast_ref):
    q = pl.program_id(1)
    fast_ref[...] = in_ref[...]
    for j in range(_SPC):
        p = _SPC * q + j
        o = (21 * p) // 5 - _CF * q
        slow_ref[:, j : j + 1] = in_ref[:, pl.ds(o, 1)]


def kernel(frames):
    B, T, H, W = frames.shape
    Ts = T // 4

    slow, fast = pl.pallas_call(
        _body,
        grid=(B, T // _CF),
        in_specs=[pl.BlockSpec((1, _CF, H, W), lambda b, q: (b, q, 0, 0))],
        out_specs=(
            pl.BlockSpec((1, _SPC, H, W), lambda b, q: (b, q, 0, 0)),
            pl.BlockSpec((1, _CF, H, W), lambda b, q: (b, q, 0, 0)),
        ),
        out_shape=(
            jax.ShapeDtypeStruct((B, Ts, H, W), frames.dtype),
            jax.ShapeDtypeStruct((B, T, H, W), frames.dtype),
        ),
        compiler_params=pltpu.CompilerParams(
            dimension_semantics=("parallel", "parallel"),
        ),
    )(frames)
    return (slow, fast)


# speedup vs baseline: 53.8353x; 1.0532x over previous
import jax
import jax.numpy as jnp
from jax.experimental import pallas as pl
from jax.experimental.pallas import tpu as pltpu

_CF = 32
_SPC = _CF // 4


def _body(in_ref, slow_ref, fast_ref):
    q = pl.program_id(1)
    fast_ref[...] = in_ref[...]
    for j in range(_SPC):
        p = _SPC * q + j
        o = (21 * p) // 5 - _CF * q
        slow_ref[:, j : j + 1] = in_ref[:, pl.ds(o, 1)]


def kernel(frames):
    B, T, H, W = frames.shape
    Ts = T // 4

    slow, fast = pl.pallas_call(
        _body,
        grid=(B, T // _CF),
        in_specs=[pl.BlockSpec((1, _CF, H, W), lambda b, q: (b, q, 0, 0))],
        out_specs=(
            pl.BlockSpec((1, _SPC, H, W), lambda b, q: (b, q, 0, 0)),
            pl.BlockSpec((1, _CF, H, W), lambda b, q: (b, q, 0, 0)),
        ),
        out_shape=(
            jax.ShapeDtypeStruct((B, Ts, H, W), frames.dtype),
            jax.ShapeDtypeStruct((B, T, H, W), frames.dtype),
        ),
        compiler_params=pltpu.CompilerParams(
            dimension_semantics=("parallel", "parallel"),
        ),
    )(frames)
    return (slow, fast)
